# baseline (speedup 1.0000x reference)
import functools

import jax
import jax.numpy as jnp
from jax import lax
from jax.experimental import pallas as pl
from jax.experimental.pallas import tpu as pltpu
from jax.experimental.pallas import tpu_sc as plsc

_ATOMS = 51
_V_MAX = 10.0
_V_MIN = -10.0
_DELTA = (_V_MAX - _V_MIN) / (_ATOMS - 1)
_BATCH = 16384

_NUM_CORES = 2
_NUM_SUBCORES = 16
_NUM_WORKERS = _NUM_CORES * _NUM_SUBCORES

_SC_ROWS = 4096
_TC_ROWS = _BATCH - _SC_ROWS
_ROWS_PER_WORKER = _SC_ROWS // _NUM_WORKERS
_WORDS_PER_WORKER = _ROWS_PER_WORKER * _ATOMS
_VECS_PER_WORKER = _WORDS_PER_WORKER // 16

_TC_BLOCKS = 4
_TC_BLOCK_ROWS = _TC_ROWS // _TC_BLOCKS

_LN2 = 0.6931471805599453


def _projection_matrix():
    supports = jnp.linspace(_V_MIN, _V_MAX, _ATOMS).astype(jnp.float32)
    tz = jnp.clip(supports, _V_MIN, _V_MAX)
    b = (tz - _V_MIN) / _DELTA
    l = jnp.floor(b).astype(jnp.int32)
    u = jnp.ceil(b).astype(jnp.int32)
    l = jnp.where((u > 0) & (l == u), l - 1, l)
    u = jnp.where((l < _ATOMS - 1) & (l == u), u + 1, u)
    wl = u.astype(jnp.float32) - b
    wu = b - l.astype(jnp.float32)
    cols = jnp.arange(_ATOMS, dtype=jnp.int32)[None, :]
    p = wl[:, None] * (l[:, None] == cols).astype(jnp.float32)
    p = p + wu[:, None] * (u[:, None] == cols).astype(jnp.float32)
    return p


def _tc_block(proj_ref, anchor_ref, feature_ref, out_ref):
    a = anchor_ref[...]
    f = feature_ref[...]
    s = jnp.dot(a, proj_ref[...], preferred_element_type=jnp.float32)
    slog = jnp.where(s == 0.0, 0.0, s * jnp.log(s))
    pointwise = slog - s * jnp.log(f + 1e-16)
    block_sum = jnp.sum(pointwise, axis=(0, 1), keepdims=True)

    @pl.when(pl.program_id(0) == 0)
    def _init():
        out_ref[...] = jnp.zeros((1, 1), jnp.float32)

    out_ref[...] += block_sum


def _tc_partial(anchor, feature):
    return pl.pallas_call(
        _tc_block,
        grid=(_TC_BLOCKS,),
        in_specs=[
            pl.BlockSpec((_ATOMS, _ATOMS), lambda i: (0, 0)),
            pl.BlockSpec((_TC_BLOCK_ROWS, _ATOMS), lambda i: (i, 0)),
            pl.BlockSpec((_TC_BLOCK_ROWS, _ATOMS), lambda i: (i, 0)),
        ],
        out_specs=pl.BlockSpec((1, 1), lambda i: (0, 0)),
        out_shape=jax.ShapeDtypeStruct((1, 1), jnp.float32),
    )(_projection_matrix(), anchor, feature)


def _log16(x):
    bits = lax.bitcast_convert_type(x, jnp.int32)
    e = ((bits >> 23) & 255) - 127
    m = lax.bitcast_convert_type(
        (bits & 0x007FFFFF) | 0x3F800000, jnp.float32)
    big = m > 1.4142135
    m = jnp.where(big, m * 0.5, m)
    e = jnp.where(big, e + 1, e)
    t = (m - 1.0) / (m + 1.0)
    t2 = t * t
    p = 1.0 + t2 * (0.33333334 + t2 * (0.2 + t2 * 0.14285715))
    return e.astype(jnp.float32) * _LN2 + 2.0 * t * p


def _sc_worker(anchor_hbm, feature_hbm, out_hbm, a_v, f_v, acc_v):
    wid = lax.axis_index("s") * _NUM_CORES + lax.axis_index("c")
    row0 = _TC_ROWS + wid * _ROWS_PER_WORKER
    pltpu.sync_copy(anchor_hbm.at[pl.ds(row0, _ROWS_PER_WORKER), :], a_v)
    pltpu.sync_copy(feature_hbm.at[pl.ds(row0, _ROWS_PER_WORKER), :], f_v)

    tail_mask = jnp.arange(16, dtype=jnp.int32) >= 13

    def term(a, f):
        t = a * (_log16(a) - _log16(f + 1e-16))
        return jnp.where(a == 0.0, 0.0, t)

    def body(r, acc):
        acc = acc + term(a_v[r, pl.ds(0, 16)], f_v[r, pl.ds(0, 16)])
        acc = acc + term(a_v[r, pl.ds(16, 16)], f_v[r, pl.ds(16, 16)])
        acc = acc + term(a_v[r, pl.ds(32, 16)], f_v[r, pl.ds(32, 16)])
        tail = term(a_v[r, pl.ds(35, 16)], f_v[r, pl.ds(35, 16)])
        return acc + jnp.where(tail_mask, tail, 0.0)

    acc = lax.fori_loop(0, _ROWS_PER_WORKER, body,
                        jnp.zeros((16,), jnp.float32))
    acc_v[...] = acc
    pltpu.sync_copy(acc_v, out_hbm.at[pl.ds(wid * 16, 16)])


def _sc_partial(anchor, feature):
    mesh = plsc.VectorSubcoreMesh(core_axis_name="c", subcore_axis_name="s")
    run = pl.kernel(
        _sc_worker,
        mesh=mesh,
        out_type=jax.ShapeDtypeStruct((_NUM_WORKERS * 16,), jnp.float32),
        scratch_types=[
            pltpu.VMEM((_ROWS_PER_WORKER, _ATOMS), jnp.float32),
            pltpu.VMEM((_ROWS_PER_WORKER, _ATOMS), jnp.float32),
            pltpu.VMEM((16,), jnp.float32),
        ],
    )
    return run(anchor, feature)


@functools.partial(jax.jit, static_argnames=())
def kernel(anchor, feature):
    sc_out = _sc_partial(anchor, feature)
    tc_out = _tc_partial(anchor, feature)
    return (tc_out[0, 0] + jnp.sum(sc_out)) / _BATCH


# speedup vs baseline: 1.0034x; 1.0034x over previous
import functools

import jax
import jax.numpy as jnp
from jax import lax
from jax.experimental import pallas as pl
from jax.experimental.pallas import tpu as pltpu
from jax.experimental.pallas import tpu_sc as plsc

_ATOMS = 51
_V_MAX = 10.0
_V_MIN = -10.0
_DELTA = (_V_MAX - _V_MIN) / (_ATOMS - 1)
_BATCH = 16384

_NUM_CORES = 2
_NUM_SUBCORES = 16
_NUM_WORKERS = _NUM_CORES * _NUM_SUBCORES

_SC_ROWS = 4096
_TC_ROWS = _BATCH - _SC_ROWS
_ROWS_PER_WORKER = _SC_ROWS // _NUM_WORKERS
_WORDS_PER_WORKER = _ROWS_PER_WORKER * _ATOMS
_VECS_PER_WORKER = _WORDS_PER_WORKER // 16

_TC_BLOCKS = 4
_TC_BLOCK_ROWS = _TC_ROWS // _TC_BLOCKS

_LN2 = 0.6931471805599453


def _projection_matrix():
    supports = jnp.linspace(_V_MIN, _V_MAX, _ATOMS).astype(jnp.float32)
    tz = jnp.clip(supports, _V_MIN, _V_MAX)
    b = (tz - _V_MIN) / _DELTA
    l = jnp.floor(b).astype(jnp.int32)
    u = jnp.ceil(b).astype(jnp.int32)
    l = jnp.where((u > 0) & (l == u), l - 1, l)
    u = jnp.where((l < _ATOMS - 1) & (l == u), u + 1, u)
    wl = u.astype(jnp.float32) - b
    wu = b - l.astype(jnp.float32)
    cols = jnp.arange(_ATOMS, dtype=jnp.int32)[None, :]
    p = wl[:, None] * (l[:, None] == cols).astype(jnp.float32)
    p = p + wu[:, None] * (u[:, None] == cols).astype(jnp.float32)
    return p


def _tc_block(proj_ref, anchor_ref, feature_ref, out_ref):
    a = anchor_ref[...]
    f = feature_ref[...]
    s = jnp.dot(a, proj_ref[...], preferred_element_type=jnp.float32)
    slog = jnp.where(s == 0.0, 0.0, s * jnp.log(s))
    pointwise = slog - s * jnp.log(f + 1e-16)
    block_sum = jnp.sum(pointwise, axis=(0, 1), keepdims=True)

    @pl.when(pl.program_id(0) == 0)
    def _init():
        out_ref[...] = jnp.zeros((1, 1), jnp.float32)

    out_ref[...] += block_sum


def _tc_partial(anchor, feature):
    return pl.pallas_call(
        _tc_block,
        grid=(_TC_BLOCKS,),
        in_specs=[
            pl.BlockSpec((_ATOMS, _ATOMS), lambda i: (0, 0)),
            pl.BlockSpec((_TC_BLOCK_ROWS, _ATOMS), lambda i: (i, 0)),
            pl.BlockSpec((_TC_BLOCK_ROWS, _ATOMS), lambda i: (i, 0)),
        ],
        out_specs=pl.BlockSpec((1, 1), lambda i: (0, 0)),
        out_shape=jax.ShapeDtypeStruct((1, 1), jnp.float32),
    )(_projection_matrix(), anchor, feature)


def _log16(x):
    bits = lax.bitcast_convert_type(x, jnp.int32)
    e = ((bits >> 23) & 255) - 127
    m = lax.bitcast_convert_type(
        (bits & 0x007FFFFF) | 0x3F800000, jnp.float32)
    big = m > 1.4142135
    m = jnp.where(big, m * 0.5, m)
    e = jnp.where(big, e + 1, e)
    t = (m - 1.0) / (m + 1.0)
    t2 = t * t
    p = 1.0 + t2 * (0.33333334 + t2 * (0.2 + t2 * 0.14285715))
    return e.astype(jnp.float32) * _LN2 + 2.0 * t * p


def _sc_worker(anchor_hbm, feature_hbm, out_hbm, a_v, f_v, acc_v):
    wid = lax.axis_index("s") * _NUM_CORES + lax.axis_index("c")
    row0 = _TC_ROWS + wid * _ROWS_PER_WORKER
    pltpu.sync_copy(anchor_hbm.at[pl.ds(row0, _ROWS_PER_WORKER), :], a_v)
    pltpu.sync_copy(feature_hbm.at[pl.ds(row0, _ROWS_PER_WORKER), :], f_v)

    tail_mask = jnp.arange(16, dtype=jnp.int32) >= 13

    def term(a, f):
        t = a * (_log16(a) - _log16(f + 1e-16))
        return jnp.where(a == 0.0, 0.0, t)

    def body(r, acc):
        acc = acc + term(a_v[r, pl.ds(0, 16)], f_v[r, pl.ds(0, 16)])
        acc = acc + term(a_v[r, pl.ds(16, 16)], f_v[r, pl.ds(16, 16)])
        acc = acc + term(a_v[r, pl.ds(32, 16)], f_v[r, pl.ds(32, 16)])
        tail = term(a_v[r, pl.ds(35, 16)], f_v[r, pl.ds(35, 16)])
        return acc + jnp.where(tail_mask, tail, 0.0)

    acc = lax.fori_loop(0, _ROWS_PER_WORKER, body,
                        jnp.zeros((16,), jnp.float32))
    acc_v[...] = acc
    pltpu.sync_copy(acc_v, out_hbm.at[pl.ds(wid * 16, 16)])


def _sc_partial(anchor, feature):
    mesh = plsc.VectorSubcoreMesh(core_axis_name="c", subcore_axis_name="s")
    run = pl.kernel(
        _sc_worker,
        mesh=mesh,
        compiler_params=pltpu.CompilerParams(use_tc_tiling_on_sc=True),
        out_type=jax.ShapeDtypeStruct((_NUM_WORKERS * 16,), jnp.float32),
        scratch_types=[
            pltpu.VMEM((_ROWS_PER_WORKER, _ATOMS), jnp.float32),
            pltpu.VMEM((_ROWS_PER_WORKER, _ATOMS), jnp.float32),
            pltpu.VMEM((16,), jnp.float32),
        ],
    )
    return run(anchor, feature)


@functools.partial(jax.jit, static_argnames=())
def kernel(anchor, feature):
    sc_out = _sc_partial(anchor, feature)
    tc_out = _tc_partial(anchor, feature)
    return (tc_out[0, 0] + jnp.sum(sc_out)) / _BATCH


# speedup vs baseline: 1.7625x; 1.7566x over previous
import functools

import jax
import jax.numpy as jnp
from jax.experimental import pallas as pl

_ATOMS = 51
_V_MAX = 10.0
_V_MIN = -10.0
_DELTA = (_V_MAX - _V_MIN) / (_ATOMS - 1)
_BATCH = 16384

_NUM_BLOCKS = 4


def _projection_matrix():
    supports = jnp.linspace(_V_MIN, _V_MAX, _ATOMS).astype(jnp.float32)
    tz = jnp.clip(supports, _V_MIN, _V_MAX)
    b = (tz - _V_MIN) / _DELTA
    l = jnp.floor(b).astype(jnp.int32)
    u = jnp.ceil(b).astype(jnp.int32)
    l = jnp.where((u > 0) & (l == u), l - 1, l)
    u = jnp.where((l < _ATOMS - 1) & (l == u), u + 1, u)
    wl = u.astype(jnp.float32) - b
    wu = b - l.astype(jnp.float32)
    cols = jnp.arange(_ATOMS, dtype=jnp.int32)[None, :]
    p = wl[:, None] * (l[:, None] == cols).astype(jnp.float32)
    p = p + wu[:, None] * (u[:, None] == cols).astype(jnp.float32)
    return p


def _kl_block(proj_ref, anchor_ref, feature_ref, out_ref):
    a = anchor_ref[...]
    f = feature_ref[...]
    s = jnp.dot(a, proj_ref[...], preferred_element_type=jnp.float32)
    slog = jnp.where(s == 0.0, 0.0, s * jnp.log(s))
    pointwise = slog - s * jnp.log(f + 1e-16)
    block_sum = jnp.sum(pointwise, axis=(0, 1), keepdims=True)

    @pl.when(pl.program_id(0) == 0)
    def _init():
        out_ref[...] = jnp.zeros((1, 1), jnp.float32)

    out_ref[...] += block_sum


@functools.partial(jax.jit, static_argnames=())
def kernel(anchor, feature):
    batch, atoms = anchor.shape
    rows = batch // _NUM_BLOCKS
    out = pl.pallas_call(
        _kl_block,
        grid=(_NUM_BLOCKS,),
        in_specs=[
            pl.BlockSpec((atoms, atoms), lambda i: (0, 0)),
            pl.BlockSpec((rows, atoms), lambda i: (i, 0)),
            pl.BlockSpec((rows, atoms), lambda i: (i, 0)),
        ],
        out_specs=pl.BlockSpec((1, 1), lambda i: (0, 0)),
        out_shape=jax.ShapeDtypeStruct((1, 1), jnp.float32),
    )(_projection_matrix(), anchor, feature)
    return out[0, 0] / batch
